# Initial kernel scaffold; baseline (speedup 1.0000x reference)
import jax
import jax.numpy as jnp
from jax.experimental import pallas as pl


def kernel(idx_x, idx_w, x_batch, edge_index, edge_weight, word_vec, W_rel, b_rel, W_root, bn1_gamma, bn1_beta, W_fc1, b_fc1, W_fc2, b_fc2, W_mean, b_mean, bn_mean_gamma, bn_mean_beta, W_logvar, b_logvar, W_phi, b_phi):
    raise NotImplementedError("write your pallas kernel here")


# speedup vs baseline: 2.6398x; 2.6398x over previous
import functools

import jax
import jax.numpy as jnp
from jax import lax
from jax.experimental import pallas as pl
from jax.experimental.pallas import tpu as pltpu
import jax.experimental.pallas.tpu_sc as plsc

N = 10000
NP = 10240
E = 160000
EP = 163840
D_IN = 256
NWID = 512
ENC_NH = 512
NT = 128
NDOC = 64
EPS = 1e-5
NC = 2
NS = 16
BLK = 1024
GRID = NP // BLK

def _sc_mesh():
    return plsc.VectorSubcoreMesh(
        core_axis_name="c", subcore_axis_name="s",
        num_cores=NC, num_subcores=NS)


def _make_sc_gather(V, D, B, K):
    bpw = B // (NC * NS)
    nchunks = bpw // K

    def body(table_hbm, idx_hbm, out_hbm, idx_v, rows_v, sem):
        wid = lax.axis_index("s") * NC + lax.axis_index("c")
        base = wid * bpw

        def chunk(j, carry):
            off = base + j * K
            pltpu.sync_copy(idx_hbm.at[pl.ds(off, K)], idx_v)
            pltpu.async_copy(table_hbm.at[idx_v], rows_v, sem).wait()
            pltpu.sync_copy(rows_v, out_hbm.at[pl.ds(off, K)])
            return carry

        lax.fori_loop(0, nchunks, chunk, 0)

    return pl.kernel(
        body,
        out_type=jax.ShapeDtypeStruct((B, D), jnp.float32),
        scratch_types=[
            pltpu.VMEM((K,), jnp.int32),
            pltpu.VMEM((K, D), jnp.float32),
            pltpu.SemaphoreType.DMA,
        ],
        mesh=_sc_mesh(),
    )


def _make_sc_scatter(K=128):
    ept = EP // NS
    nchunks = ept // K
    zrows = NP // NS

    def body(msg_hbm, dst_hbm, zeros_hbm, out_hbm, idx_v, rows_v, acc, sem):
        c = lax.axis_index("c")
        s = lax.axis_index("s")
        pltpu.sync_copy(zeros_hbm.at[pl.ds(s * zrows, zrows)],
                        acc.at[pl.ds(s * zrows, zrows)])
        plsc.subcore_barrier()
        base = s * ept

        def chunk(j, carry):
            off = base + j * K
            pltpu.sync_copy(dst_hbm.at[pl.ds(off, K)], idx_v)
            pltpu.sync_copy(msg_hbm.at[pl.ds(off, K), pl.ds(c * 128, 128)],
                            rows_v)
            pltpu.sync_copy(rows_v, acc.at[idx_v], add=True)
            return carry

        lax.fori_loop(0, nchunks, chunk, 0)
        plsc.subcore_barrier()
        pltpu.sync_copy(acc.at[pl.ds(s * zrows, zrows)],
                        out_hbm.at[pl.ds(s * zrows, zrows),
                                   pl.ds(c * 128, 128)])

    return pl.kernel(
        body,
        out_type=jax.ShapeDtypeStruct((NP, 2 * 128), jnp.float32),
        scratch_types=[
            pltpu.VMEM((K,), jnp.int32),
            pltpu.VMEM((K, 128), jnp.float32),
            pltpu.VMEM_SHARED((NP, 128), jnp.float32),
            pltpu.SemaphoreType.DMA,
        ],
        mesh=_sc_mesh(),
    )


def _scale_body(m_ref, w_ref, o_ref):
    o_ref[...] = m_ref[...] * w_ref[...]


def _tc_scale(msg, ew):
    blk = 2048
    return pl.pallas_call(
        _scale_body,
        grid=(EP // blk,),
        in_specs=[
            pl.BlockSpec((blk, D_IN), lambda i: (i, 0)),
            pl.BlockSpec((blk, 1), lambda i: (i, 0)),
        ],
        out_specs=pl.BlockSpec((blk, D_IN), lambda i: (i, 0)),
        out_shape=jax.ShapeDtypeStruct((EP, D_IN), jnp.float32),
    )(msg, ew)


def _bdot(a, b):
    return jnp.dot(a.astype(jnp.bfloat16), b.astype(jnp.bfloat16),
                   preferred_element_type=jnp.float32)


def _h_body(agg_ref, x_ref, iw_ref, wrel_ref, wroot_ref, brel_ref,
            h_ref, st_ref):
    i = pl.program_id(0)
    x = x_ref[...]
    agg = agg_ref[...] + iw_ref[...] * x
    h = _bdot(agg, wrel_ref[...]) + _bdot(x, wroot_ref[...]) + brel_ref[...]
    h_ref[...] = h
    gid = i * BLK + lax.broadcasted_iota(jnp.int32, (BLK, 1), 0)
    mf = (gid < N).astype(jnp.float32)
    hm = h * mf
    s0 = jnp.sum(hm, axis=0, keepdims=True)
    s1 = jnp.sum(h * hm, axis=0, keepdims=True)
    blkstats = jnp.concatenate(
        [s0, s1, jnp.zeros((6, NWID), jnp.float32)], axis=0)

    @pl.when(i == 0)
    def _():
        st_ref[...] = jnp.zeros_like(st_ref)

    st_ref[...] += blkstats


def _tc_h():
    return pl.pallas_call(
        _h_body,
        grid=(GRID,),
        in_specs=[
            pl.BlockSpec((BLK, D_IN), lambda i: (i, 0)),
            pl.BlockSpec((BLK, D_IN), lambda i: (i, 0)),
            pl.BlockSpec((BLK, 1), lambda i: (i, 0)),
            pl.BlockSpec((D_IN, NWID), lambda i: (0, 0)),
            pl.BlockSpec((D_IN, NWID), lambda i: (0, 0)),
            pl.BlockSpec((1, NWID), lambda i: (0, 0)),
        ],
        out_specs=[
            pl.BlockSpec((BLK, NWID), lambda i: (i, 0)),
            pl.BlockSpec((8, NWID), lambda i: (0, 0)),
        ],
        out_shape=[
            jax.ShapeDtypeStruct((NP, NWID), jnp.float32),
            jax.ShapeDtypeStruct((8, NWID), jnp.float32),
        ],
    )


def _enc_body(h_ref, st_ref, x_ref, xb_ref, g1_ref, b1_ref,
              wf1_ref, bf1_ref, wf2_ref, bf2_ref,
              enc1_ref, doc_ref):
    i = pl.program_id(0)
    m = st_ref[0, :] * (1.0 / N)
    v = st_ref[1, :] * (1.0 / N) - m * m
    inv = lax.rsqrt(v + EPS)
    t = jnp.tanh((h_ref[...] - m[None, :]) * inv[None, :] * g1_ref[...]
                 + b1_ref[...])
    enc1 = jnp.concatenate([t, x_ref[...]], axis=1)
    enc1_ref[...] = enc1
    a = jax.nn.sigmoid(_bdot(enc1, wf1_ref[...]) + bf1_ref[...])
    b = jnp.tanh(_bdot(enc1, wf2_ref[...]) + bf2_ref[...])
    enc2 = a * b
    oh = (xb_ref[...] ==
          lax.broadcasted_iota(jnp.int32, (1, NDOC), 1)).astype(jnp.float32)
    part = lax.dot_general(oh, enc2, (((0,), (0,)), ((), ())),
                           preferred_element_type=jnp.float32, precision=lax.Precision.HIGHEST)

    @pl.when(i == 0)
    def _():
        doc_ref[...] = jnp.zeros_like(doc_ref)

    doc_ref[...] += part


def _tc_enc():
    return pl.pallas_call(
        _enc_body,
        grid=(GRID,),
        in_specs=[
            pl.BlockSpec((BLK, NWID), lambda i: (i, 0)),
            pl.BlockSpec((8, NWID), lambda i: (0, 0)),
            pl.BlockSpec((BLK, D_IN), lambda i: (i, 0)),
            pl.BlockSpec((BLK, 1), lambda i: (i, 0)),
            pl.BlockSpec((1, NWID), lambda i: (0, 0)),
            pl.BlockSpec((1, NWID), lambda i: (0, 0)),
            pl.BlockSpec((NWID + D_IN, ENC_NH), lambda i: (0, 0)),
            pl.BlockSpec((1, ENC_NH), lambda i: (0, 0)),
            pl.BlockSpec((NWID + D_IN, ENC_NH), lambda i: (0, 0)),
            pl.BlockSpec((1, ENC_NH), lambda i: (0, 0)),
        ],
        out_specs=[
            pl.BlockSpec((BLK, NWID + D_IN), lambda i: (i, 0)),
            pl.BlockSpec((NDOC, ENC_NH), lambda i: (0, 0)),
        ],
        out_shape=[
            jax.ShapeDtypeStruct((NP, NWID + D_IN), jnp.float32),
            jax.ShapeDtypeStruct((NDOC, ENC_NH), jnp.float32),
        ],
    )


def _doc_body(d_ref, wm_ref, bm_ref, gm_ref, btm_ref, wl_ref, bl_ref,
              wpb_ref, mean_ref, logvar_ref, dp_ref):
    d = d_ref[...]
    mp = _bdot(d, wm_ref[...]) + bm_ref[...]
    mm = jnp.mean(mp, axis=0, keepdims=True)
    vv = jnp.mean(mp * mp, axis=0, keepdims=True) - mm * mm
    mean_ref[...] = (mp - mm) * lax.rsqrt(vv + EPS) * gm_ref[...] + btm_ref[...]
    logvar_ref[...] = _bdot(d, wl_ref[...]) + bl_ref[...]
    dp_ref[...] = _bdot(d, wpb_ref[...])


def _tc_doc():
    return pl.pallas_call(
        _doc_body,
        out_shape=[
            jax.ShapeDtypeStruct((NDOC, NT), jnp.float32),
            jax.ShapeDtypeStruct((NDOC, NT), jnp.float32),
            jax.ShapeDtypeStruct((NDOC, NT), jnp.float32),
        ],
    )


def _phi_body(enc1_ref, xb_ref, dp_ref, wpa_ref, bp_ref, phi_ref):
    oh = (xb_ref[...] ==
          lax.broadcasted_iota(jnp.int32, (1, NDOC), 1)).astype(jnp.float32)
    logits = (_bdot(enc1_ref[...], wpa_ref[...])
              + jnp.dot(oh, dp_ref[...], preferred_element_type=jnp.float32,
                        precision=lax.Precision.HIGHEST)
              + bp_ref[...])
    z = logits - jnp.max(logits, axis=1, keepdims=True)
    ez = jnp.exp(z)
    phi_ref[...] = ez / jnp.sum(ez, axis=1, keepdims=True)


def _tc_phi():
    return pl.pallas_call(
        _phi_body,
        grid=(GRID,),
        in_specs=[
            pl.BlockSpec((BLK, NWID + D_IN), lambda i: (i, 0)),
            pl.BlockSpec((BLK, 1), lambda i: (i, 0)),
            pl.BlockSpec((NDOC, NT), lambda i: (0, 0)),
            pl.BlockSpec((NWID + D_IN, NT), lambda i: (0, 0)),
            pl.BlockSpec((1, NT), lambda i: (0, 0)),
        ],
        out_specs=pl.BlockSpec((BLK, NT), lambda i: (i, 0)),
        out_shape=jax.ShapeDtypeStruct((NP, NT), jnp.float32),
    )


def kernel(idx_x, idx_w, x_batch, edge_index, edge_weight, word_vec,
           W_rel, b_rel, W_root, bn1_gamma, bn1_beta,
           W_fc1, b_fc1, W_fc2, b_fc2,
           W_mean, b_mean, bn_mean_gamma, bn_mean_beta,
           W_logvar, b_logvar, W_phi, b_phi):
    f32 = jnp.float32
    idx_pad = jnp.pad(idx_x.astype(jnp.int32), (0, NP - N))
    src = jnp.pad(edge_index[0].astype(jnp.int32), (0, EP - E))
    dst = jnp.pad(edge_index[1].astype(jnp.int32), (0, EP - E))
    ew = jnp.pad(edge_weight, (0, EP - E)).reshape(EP, 1)
    iw = jnp.pad(idx_w, (0, NP - N)).reshape(NP, 1)
    xb = jnp.pad(x_batch, (0, NP - N), constant_values=NDOC).reshape(NP, 1)
    zeros_half = jnp.zeros((NP, 128), f32)

    x = _make_sc_gather(50000, D_IN, NP, 160)(word_vec, idx_pad)
    msg = _make_sc_gather(NP, D_IN, EP, 128)(x, src)
    msg_s = _tc_scale(msg, ew)
    agg = _make_sc_scatter()(msg_s, dst, zeros_half)

    h, stats = _tc_h()(
        agg, x, iw, W_rel, W_root, b_rel.reshape(1, NWID))
    enc1, doc = _tc_enc()(
        h, stats, x, xb, bn1_gamma.reshape(1, NWID), bn1_beta.reshape(1, NWID),
        W_fc1, b_fc1.reshape(1, ENC_NH), W_fc2, b_fc2.reshape(1, ENC_NH))
    mean, logvar, docproj = _tc_doc()(
        doc, W_mean, b_mean.reshape(1, NT), bn_mean_gamma.reshape(1, NT),
        bn_mean_beta.reshape(1, NT), W_logvar, b_logvar.reshape(1, NT),
        W_phi[NWID + D_IN:, :])
    phi = _tc_phi()(enc1, xb, docproj, W_phi[:NWID + D_IN, :],
                    b_phi.reshape(1, NT))
    return (mean, logvar, phi[:N])


# speedup vs baseline: 2.9942x; 1.1343x over previous
import functools

import jax
import jax.numpy as jnp
from jax import lax
from jax.experimental import pallas as pl
from jax.experimental.pallas import tpu as pltpu
import jax.experimental.pallas.tpu_sc as plsc

N = 10000
NP = 10240
E = 160000
EP = 163840
D_IN = 256
NWID = 512
ENC_NH = 512
NT = 128
NDOC = 64
EPS = 1e-5
NC = 2
NS = 16
BLK = 1024
GRID = NP // BLK

def _sc_mesh():
    return plsc.VectorSubcoreMesh(
        core_axis_name="c", subcore_axis_name="s",
        num_cores=NC, num_subcores=NS)


def _make_sc_gather(V, D, B, K):
    bpw = B // (NC * NS)
    nchunks = bpw // K

    def body(table_hbm, idx_hbm, out_hbm, idx_v, rows_v, sem):
        wid = lax.axis_index("s") * NC + lax.axis_index("c")
        base = wid * bpw

        def chunk(j, carry):
            off = base + j * K
            pltpu.sync_copy(idx_hbm.at[pl.ds(off, K)], idx_v)
            pltpu.async_copy(table_hbm.at[idx_v], rows_v, sem).wait()
            pltpu.sync_copy(rows_v, out_hbm.at[pl.ds(off, K)])
            return carry

        lax.fori_loop(0, nchunks, chunk, 0)

    return pl.kernel(
        body,
        out_type=jax.ShapeDtypeStruct((B, D), jnp.float32),
        scratch_types=[
            pltpu.VMEM((K,), jnp.int32),
            pltpu.VMEM((K, D), jnp.float32),
            pltpu.SemaphoreType.DMA,
        ],
        mesh=_sc_mesh(),
    )


def _make_sc_edge(K=128):
    nchunks_total = EP // K
    ntile = nchunks_total // NS
    zrows = NP // NS

    def body(xv_hbm, epk_hbm, ewr_hbm, zeros_hbm, out_hbm,
             ebuf, ewb, rows_v, acc, sem):
        c = lax.axis_index("c")
        s = lax.axis_index("s")
        pltpu.sync_copy(zeros_hbm.at[pl.ds(s * zrows, zrows)],
                        acc.at[pl.ds(s * zrows, zrows)])
        plsc.subcore_barrier()

        def chunk(j, carry):
            cid = s * ntile + j
            pltpu.sync_copy(epk_hbm.at[cid], ebuf)
            pltpu.sync_copy(ewr_hbm.at[cid], ewb)
            pltpu.async_copy(xv_hbm.at[ebuf.at[0], pl.ds(c, 1), :],
                             rows_v, sem).wait()

            def edge(e, carry2):
                ew_bc = ewb[e, pl.ds(0, 16)]
                for jj in range(8):
                    sl = pl.ds(jj * 16, 16)
                    rows_v[e, 0, sl] = rows_v[e, 0, sl] * ew_bc
                return carry2

            lax.fori_loop(0, K, edge, 0)
            pltpu.sync_copy(rows_v, acc.at[ebuf.at[1]], add=True)
            return carry

        lax.fori_loop(0, ntile, chunk, 0)
        plsc.subcore_barrier()
        pltpu.sync_copy(acc.at[pl.ds(s * zrows, zrows)],
                        out_hbm.at[pl.ds(s * zrows, zrows), pl.ds(c, 1), :])

    return pl.kernel(
        body,
        out_type=jax.ShapeDtypeStruct((NP, 2, 128), jnp.float32),
        scratch_types=[
            pltpu.VMEM((2, K), jnp.int32),
            pltpu.VMEM((K, 16), jnp.float32),
            pltpu.VMEM((K, 1, 128), jnp.float32),
            pltpu.VMEM_SHARED((NP, 1, 128), jnp.float32),
            pltpu.SemaphoreType.DMA,
        ],
        mesh=_sc_mesh(),
    )


def _bdot(a, b):
    return jnp.dot(a.astype(jnp.bfloat16), b.astype(jnp.bfloat16),
                   preferred_element_type=jnp.float32)


def _h_body(agg_ref, x_ref, iw_ref, wrel_ref, wroot_ref, brel_ref,
            h_ref, st_ref):
    i = pl.program_id(0)
    x = x_ref[...]
    agg = agg_ref[...] + iw_ref[...] * x
    h = _bdot(agg, wrel_ref[...]) + _bdot(x, wroot_ref[...]) + brel_ref[...]
    h_ref[...] = h
    gid = i * BLK + lax.broadcasted_iota(jnp.int32, (BLK, 1), 0)
    mf = (gid < N).astype(jnp.float32)
    hm = h * mf
    s0 = jnp.sum(hm, axis=0, keepdims=True)
    s1 = jnp.sum(h * hm, axis=0, keepdims=True)
    blkstats = jnp.concatenate(
        [s0, s1, jnp.zeros((6, NWID), jnp.float32)], axis=0)

    @pl.when(i == 0)
    def _():
        st_ref[...] = jnp.zeros_like(st_ref)

    st_ref[...] += blkstats


def _tc_h():
    return pl.pallas_call(
        _h_body,
        grid=(GRID,),
        in_specs=[
            pl.BlockSpec((BLK, D_IN), lambda i: (i, 0)),
            pl.BlockSpec((BLK, D_IN), lambda i: (i, 0)),
            pl.BlockSpec((BLK, 1), lambda i: (i, 0)),
            pl.BlockSpec((D_IN, NWID), lambda i: (0, 0)),
            pl.BlockSpec((D_IN, NWID), lambda i: (0, 0)),
            pl.BlockSpec((1, NWID), lambda i: (0, 0)),
        ],
        out_specs=[
            pl.BlockSpec((BLK, NWID), lambda i: (i, 0)),
            pl.BlockSpec((8, NWID), lambda i: (0, 0)),
        ],
        out_shape=[
            jax.ShapeDtypeStruct((NP, NWID), jnp.float32),
            jax.ShapeDtypeStruct((8, NWID), jnp.float32),
        ],
    )


def _enc_body(h_ref, st_ref, x_ref, xb_ref, g1_ref, b1_ref,
              wf1_ref, bf1_ref, wf2_ref, bf2_ref,
              enc1_ref, doc_ref):
    i = pl.program_id(0)
    m = st_ref[0, :] * (1.0 / N)
    v = st_ref[1, :] * (1.0 / N) - m * m
    inv = lax.rsqrt(v + EPS)
    t = jnp.tanh((h_ref[...] - m[None, :]) * inv[None, :] * g1_ref[...]
                 + b1_ref[...])
    enc1 = jnp.concatenate([t, x_ref[...]], axis=1)
    enc1_ref[...] = enc1
    a = jax.nn.sigmoid(_bdot(enc1, wf1_ref[...]) + bf1_ref[...])
    b = jnp.tanh(_bdot(enc1, wf2_ref[...]) + bf2_ref[...])
    enc2 = a * b
    oh = (xb_ref[...] ==
          lax.broadcasted_iota(jnp.int32, (1, NDOC), 1)).astype(jnp.float32)
    part = lax.dot_general(oh, enc2, (((0,), (0,)), ((), ())),
                           preferred_element_type=jnp.float32, precision=lax.Precision.HIGHEST)

    @pl.when(i == 0)
    def _():
        doc_ref[...] = jnp.zeros_like(doc_ref)

    doc_ref[...] += part


def _tc_enc():
    return pl.pallas_call(
        _enc_body,
        grid=(GRID,),
        in_specs=[
            pl.BlockSpec((BLK, NWID), lambda i: (i, 0)),
            pl.BlockSpec((8, NWID), lambda i: (0, 0)),
            pl.BlockSpec((BLK, D_IN), lambda i: (i, 0)),
            pl.BlockSpec((BLK, 1), lambda i: (i, 0)),
            pl.BlockSpec((1, NWID), lambda i: (0, 0)),
            pl.BlockSpec((1, NWID), lambda i: (0, 0)),
            pl.BlockSpec((NWID + D_IN, ENC_NH), lambda i: (0, 0)),
            pl.BlockSpec((1, ENC_NH), lambda i: (0, 0)),
            pl.BlockSpec((NWID + D_IN, ENC_NH), lambda i: (0, 0)),
            pl.BlockSpec((1, ENC_NH), lambda i: (0, 0)),
        ],
        out_specs=[
            pl.BlockSpec((BLK, NWID + D_IN), lambda i: (i, 0)),
            pl.BlockSpec((NDOC, ENC_NH), lambda i: (0, 0)),
        ],
        out_shape=[
            jax.ShapeDtypeStruct((NP, NWID + D_IN), jnp.float32),
            jax.ShapeDtypeStruct((NDOC, ENC_NH), jnp.float32),
        ],
    )


def _doc_body(d_ref, wm_ref, bm_ref, gm_ref, btm_ref, wl_ref, bl_ref,
              wpb_ref, mean_ref, logvar_ref, dp_ref):
    d = d_ref[...]
    mp = _bdot(d, wm_ref[...]) + bm_ref[...]
    mm = jnp.mean(mp, axis=0, keepdims=True)
    vv = jnp.mean(mp * mp, axis=0, keepdims=True) - mm * mm
    mean_ref[...] = (mp - mm) * lax.rsqrt(vv + EPS) * gm_ref[...] + btm_ref[...]
    logvar_ref[...] = _bdot(d, wl_ref[...]) + bl_ref[...]
    dp_ref[...] = _bdot(d, wpb_ref[...])


def _tc_doc():
    return pl.pallas_call(
        _doc_body,
        out_shape=[
            jax.ShapeDtypeStruct((NDOC, NT), jnp.float32),
            jax.ShapeDtypeStruct((NDOC, NT), jnp.float32),
            jax.ShapeDtypeStruct((NDOC, NT), jnp.float32),
        ],
    )


def _phi_body(enc1_ref, xb_ref, dp_ref, wpa_ref, bp_ref, phi_ref):
    oh = (xb_ref[...] ==
          lax.broadcasted_iota(jnp.int32, (1, NDOC), 1)).astype(jnp.float32)
    logits = (_bdot(enc1_ref[...], wpa_ref[...])
              + jnp.dot(oh, dp_ref[...], preferred_element_type=jnp.float32,
                        precision=lax.Precision.HIGHEST)
              + bp_ref[...])
    z = logits - jnp.max(logits, axis=1, keepdims=True)
    ez = jnp.exp(z)
    phi_ref[...] = ez / jnp.sum(ez, axis=1, keepdims=True)


def _tc_phi():
    return pl.pallas_call(
        _phi_body,
        grid=(GRID,),
        in_specs=[
            pl.BlockSpec((BLK, NWID + D_IN), lambda i: (i, 0)),
            pl.BlockSpec((BLK, 1), lambda i: (i, 0)),
            pl.BlockSpec((NDOC, NT), lambda i: (0, 0)),
            pl.BlockSpec((NWID + D_IN, NT), lambda i: (0, 0)),
            pl.BlockSpec((1, NT), lambda i: (0, 0)),
        ],
        out_specs=pl.BlockSpec((BLK, NT), lambda i: (i, 0)),
        out_shape=jax.ShapeDtypeStruct((NP, NT), jnp.float32),
    )


def kernel(idx_x, idx_w, x_batch, edge_index, edge_weight, word_vec,
           W_rel, b_rel, W_root, bn1_gamma, bn1_beta,
           W_fc1, b_fc1, W_fc2, b_fc2,
           W_mean, b_mean, bn_mean_gamma, bn_mean_beta,
           W_logvar, b_logvar, W_phi, b_phi):
    f32 = jnp.float32
    idx_pad = jnp.pad(idx_x.astype(jnp.int32), (0, NP - N))
    src = jnp.pad(edge_index[0].astype(jnp.int32), (0, EP - E))
    dst = jnp.pad(edge_index[1].astype(jnp.int32), (0, EP - E))
    ew_pad = jnp.pad(edge_weight, (0, EP - E))
    epk = jnp.stack([src.reshape(-1, 128), dst.reshape(-1, 128)],
                    axis=1)
    ew_rep = jnp.broadcast_to(ew_pad.reshape(EP // 128, 128, 1),
                              (EP // 128, 128, 16))
    iw = jnp.pad(idx_w, (0, NP - N)).reshape(NP, 1)
    xb = jnp.pad(x_batch, (0, NP - N), constant_values=NDOC).reshape(NP, 1)
    zeros_half = jnp.zeros((NP, 1, 128), f32)

    x = _make_sc_gather(50000, D_IN, NP, 160)(word_vec, idx_pad)
    agg = _make_sc_edge()(x.reshape(NP, 2, 128), epk, ew_rep,
                          zeros_half).reshape(NP, D_IN)

    h, stats = _tc_h()(
        agg, x, iw, W_rel, W_root, b_rel.reshape(1, NWID))
    enc1, doc = _tc_enc()(
        h, stats, x, xb, bn1_gamma.reshape(1, NWID), bn1_beta.reshape(1, NWID),
        W_fc1, b_fc1.reshape(1, ENC_NH), W_fc2, b_fc2.reshape(1, ENC_NH))
    mean, logvar, docproj = _tc_doc()(
        doc, W_mean, b_mean.reshape(1, NT), bn_mean_gamma.reshape(1, NT),
        bn_mean_beta.reshape(1, NT), W_logvar, b_logvar.reshape(1, NT),
        W_phi[NWID + D_IN:, :])
    phi = _tc_phi()(enc1, xb, docproj, W_phi[:NWID + D_IN, :],
                    b_phi.reshape(1, NT))
    return (mean, logvar, phi[:N])


# speedup vs baseline: 3.6020x; 1.2030x over previous
import functools

import jax
import jax.numpy as jnp
from jax import lax
from jax.experimental import pallas as pl
from jax.experimental.pallas import tpu as pltpu
import jax.experimental.pallas.tpu_sc as plsc

N = 10000
NP = 10240
E = 160000
EP = 163840
D_IN = 256
NWID = 512
ENC_NH = 512
NT = 128
NDOC = 64
EPS = 1e-5
NC = 2
NS = 16
EDGE_K = 80
BLK = 1024
GRID = NP // BLK

def _sc_mesh():
    return plsc.VectorSubcoreMesh(
        core_axis_name="c", subcore_axis_name="s",
        num_cores=NC, num_subcores=NS)


def _make_sc_gather(V, D, B, K):
    bpw = B // (NC * NS)
    nchunks = bpw // K

    def body(table_hbm, idx_hbm, out_hbm, idx_v, rows_v, sem):
        wid = lax.axis_index("s") * NC + lax.axis_index("c")
        base = wid * bpw

        def chunk(j, carry):
            off = base + j * K
            pltpu.sync_copy(idx_hbm.at[pl.ds(off, K)], idx_v)
            pltpu.async_copy(table_hbm.at[idx_v], rows_v, sem).wait()
            pltpu.sync_copy(rows_v, out_hbm.at[pl.ds(off, K)])
            return carry

        lax.fori_loop(0, nchunks, chunk, 0)

    return pl.kernel(
        body,
        out_type=jax.ShapeDtypeStruct((B, D), jnp.float32),
        scratch_types=[
            pltpu.VMEM((K,), jnp.int32),
            pltpu.VMEM((K, D), jnp.float32),
            pltpu.SemaphoreType.DMA,
        ],
        mesh=_sc_mesh(),
    )


def _make_sc_edge(K=EDGE_K):
    nchunks_total = EP // K
    ntile = nchunks_total // NS
    zrows = NP // NS

    def body(xv_hbm, epk_hbm, ewr_hbm, zeros_hbm, out_hbm,
             ebuf, ewb, rows, dstb, acc, isem0, isem1, gsem0, gsem1):
        c = lax.axis_index("c")
        s = lax.axis_index("s")
        isem = (isem0, isem1)
        gsem = (gsem0, gsem1)
        pltpu.sync_copy(zeros_hbm.at[pl.ds(s * zrows, zrows)],
                        acc.at[pl.ds(s * zrows, zrows)])
        plsc.subcore_barrier()
        base = s * ntile

        def issue_idx(cid, sl):
            pltpu.async_copy(epk_hbm.at[cid], ebuf.at[sl], isem[sl])
            pltpu.async_copy(ewr_hbm.at[cid], ewb.at[sl], isem[sl])

        def wait_idx(sl):
            pltpu.make_async_copy(epk_hbm.at[0], ebuf.at[sl], isem[sl]).wait()
            pltpu.make_async_copy(ewr_hbm.at[0], ewb.at[sl], isem[sl]).wait()

        def issue_gather(sl):
            pltpu.async_copy(xv_hbm.at[ebuf.at[sl, 0], pl.ds(c, 1), :],
                             rows.at[sl], gsem[sl])

        def wait_gather(sl):
            pltpu.make_async_copy(xv_hbm.at[ebuf.at[sl, 0], pl.ds(c, 1), :],
                                  rows.at[sl], gsem[sl]).wait()

        def scale(sl):
            def quad(q, carry):
                e0 = q * 4
                for u in range(4):
                    e = e0 + u
                    ew_bc = ewb[sl, e, pl.ds(0, 16)]
                    for jj in range(8):
                        cs = pl.ds(jj * 16, 16)
                        rows[sl, e, 0, cs] = rows[sl, e, 0, cs] * ew_bc
                return carry
            lax.fori_loop(0, K // 4, quad, 0)

        issue_idx(base, 0)
        issue_idx(base + 1, 1)
        wait_idx(0)
        issue_gather(0)

        def pair(jj, carry):
            j0 = 2 * jj
            for sl in (0, 1):
                j = j0 + sl
                o = 1 - sl
                wait_idx(o)
                issue_gather(o)
                wait_gather(sl)
                scale(sl)
                for q in range(K // 16):
                    qs = pl.ds(q * 16, 16)
                    dstb[qs] = ebuf[sl, 1, qs]
                issue_idx(base + j + 2, sl)
                pltpu.sync_copy(rows.at[sl], acc.at[dstb], add=True)
            return carry

        lax.fori_loop(0, ntile // 2, pair, 0)
        wait_idx(1)
        wait_gather(0)
        plsc.subcore_barrier()
        pltpu.sync_copy(acc.at[pl.ds(s * zrows, zrows)],
                        out_hbm.at[pl.ds(s * zrows, zrows), pl.ds(c, 1), :])

    return pl.kernel(
        body,
        out_type=jax.ShapeDtypeStruct((NP, 2, 128), jnp.float32),
        scratch_types=[
            pltpu.VMEM((2, 2, K), jnp.int32),
            pltpu.VMEM((2, K, 16), jnp.float32),
            pltpu.VMEM((2, K, 1, 128), jnp.float32),
            pltpu.VMEM((K,), jnp.int32),
            pltpu.VMEM_SHARED((NP, 1, 128), jnp.float32),
            pltpu.SemaphoreType.DMA,
            pltpu.SemaphoreType.DMA,
            pltpu.SemaphoreType.DMA,
            pltpu.SemaphoreType.DMA,
        ],
        mesh=_sc_mesh(),
    )


def _bdot(a, b):
    return jnp.dot(a.astype(jnp.bfloat16), b.astype(jnp.bfloat16),
                   preferred_element_type=jnp.float32)


def _h_body(agg_ref, x_ref, iw_ref, wrel_ref, wroot_ref, brel_ref,
            h_ref, st_ref):
    i = pl.program_id(0)
    x = x_ref[...]
    agg = agg_ref[...] + iw_ref[...] * x
    h = _bdot(agg, wrel_ref[...]) + _bdot(x, wroot_ref[...]) + brel_ref[...]
    h_ref[...] = h
    gid = i * BLK + lax.broadcasted_iota(jnp.int32, (BLK, 1), 0)
    mf = (gid < N).astype(jnp.float32)
    hm = h * mf
    s0 = jnp.sum(hm, axis=0, keepdims=True)
    s1 = jnp.sum(h * hm, axis=0, keepdims=True)
    blkstats = jnp.concatenate(
        [s0, s1, jnp.zeros((6, NWID), jnp.float32)], axis=0)

    @pl.when(i == 0)
    def _():
        st_ref[...] = jnp.zeros_like(st_ref)

    st_ref[...] += blkstats


def _tc_h():
    return pl.pallas_call(
        _h_body,
        grid=(GRID,),
        in_specs=[
            pl.BlockSpec((BLK, D_IN), lambda i: (i, 0)),
            pl.BlockSpec((BLK, D_IN), lambda i: (i, 0)),
            pl.BlockSpec((BLK, 1), lambda i: (i, 0)),
            pl.BlockSpec((D_IN, NWID), lambda i: (0, 0)),
            pl.BlockSpec((D_IN, NWID), lambda i: (0, 0)),
            pl.BlockSpec((1, NWID), lambda i: (0, 0)),
        ],
        out_specs=[
            pl.BlockSpec((BLK, NWID), lambda i: (i, 0)),
            pl.BlockSpec((8, NWID), lambda i: (0, 0)),
        ],
        out_shape=[
            jax.ShapeDtypeStruct((NP, NWID), jnp.float32),
            jax.ShapeDtypeStruct((8, NWID), jnp.float32),
        ],
    )


def _enc_body(h_ref, st_ref, x_ref, xb_ref, g1_ref, b1_ref,
              wf1_ref, bf1_ref, wf2_ref, bf2_ref,
              enc1_ref, doc_ref):
    i = pl.program_id(0)
    m = st_ref[0, :] * (1.0 / N)
    v = st_ref[1, :] * (1.0 / N) - m * m
    inv = lax.rsqrt(v + EPS)
    t = jnp.tanh((h_ref[...] - m[None, :]) * inv[None, :] * g1_ref[...]
                 + b1_ref[...])
    enc1 = jnp.concatenate([t, x_ref[...]], axis=1)
    enc1_ref[...] = enc1
    a = jax.nn.sigmoid(_bdot(enc1, wf1_ref[...]) + bf1_ref[...])
    b = jnp.tanh(_bdot(enc1, wf2_ref[...]) + bf2_ref[...])
    enc2 = a * b
    oh = (xb_ref[...] ==
          lax.broadcasted_iota(jnp.int32, (1, NDOC), 1)).astype(jnp.float32)
    part = lax.dot_general(oh, enc2, (((0,), (0,)), ((), ())),
                           preferred_element_type=jnp.float32, precision=lax.Precision.HIGHEST)

    @pl.when(i == 0)
    def _():
        doc_ref[...] = jnp.zeros_like(doc_ref)

    doc_ref[...] += part


def _tc_enc():
    return pl.pallas_call(
        _enc_body,
        grid=(GRID,),
        in_specs=[
            pl.BlockSpec((BLK, NWID), lambda i: (i, 0)),
            pl.BlockSpec((8, NWID), lambda i: (0, 0)),
            pl.BlockSpec((BLK, D_IN), lambda i: (i, 0)),
            pl.BlockSpec((BLK, 1), lambda i: (i, 0)),
            pl.BlockSpec((1, NWID), lambda i: (0, 0)),
            pl.BlockSpec((1, NWID), lambda i: (0, 0)),
            pl.BlockSpec((NWID + D_IN, ENC_NH), lambda i: (0, 0)),
            pl.BlockSpec((1, ENC_NH), lambda i: (0, 0)),
            pl.BlockSpec((NWID + D_IN, ENC_NH), lambda i: (0, 0)),
            pl.BlockSpec((1, ENC_NH), lambda i: (0, 0)),
        ],
        out_specs=[
            pl.BlockSpec((BLK, NWID + D_IN), lambda i: (i, 0)),
            pl.BlockSpec((NDOC, ENC_NH), lambda i: (0, 0)),
        ],
        out_shape=[
            jax.ShapeDtypeStruct((NP, NWID + D_IN), jnp.float32),
            jax.ShapeDtypeStruct((NDOC, ENC_NH), jnp.float32),
        ],
    )


def _doc_body(d_ref, wm_ref, bm_ref, gm_ref, btm_ref, wl_ref, bl_ref,
              wpb_ref, mean_ref, logvar_ref, dp_ref):
    d = d_ref[...]
    mp = _bdot(d, wm_ref[...]) + bm_ref[...]
    mm = jnp.mean(mp, axis=0, keepdims=True)
    vv = jnp.mean(mp * mp, axis=0, keepdims=True) - mm * mm
    mean_ref[...] = (mp - mm) * lax.rsqrt(vv + EPS) * gm_ref[...] + btm_ref[...]
    logvar_ref[...] = _bdot(d, wl_ref[...]) + bl_ref[...]
    dp_ref[...] = _bdot(d, wpb_ref[...])


def _tc_doc():
    return pl.pallas_call(
        _doc_body,
        out_shape=[
            jax.ShapeDtypeStruct((NDOC, NT), jnp.float32),
            jax.ShapeDtypeStruct((NDOC, NT), jnp.float32),
            jax.ShapeDtypeStruct((NDOC, NT), jnp.float32),
        ],
    )


def _phi_body(enc1_ref, xb_ref, dp_ref, wpa_ref, bp_ref, phi_ref):
    oh = (xb_ref[...] ==
          lax.broadcasted_iota(jnp.int32, (1, NDOC), 1)).astype(jnp.float32)
    logits = (_bdot(enc1_ref[...], wpa_ref[...])
              + jnp.dot(oh, dp_ref[...], preferred_element_type=jnp.float32,
                        precision=lax.Precision.HIGHEST)
              + bp_ref[...])
    z = logits - jnp.max(logits, axis=1, keepdims=True)
    ez = jnp.exp(z)
    phi_ref[...] = ez / jnp.sum(ez, axis=1, keepdims=True)


def _tc_phi():
    return pl.pallas_call(
        _phi_body,
        grid=(GRID,),
        in_specs=[
            pl.BlockSpec((BLK, NWID + D_IN), lambda i: (i, 0)),
            pl.BlockSpec((BLK, 1), lambda i: (i, 0)),
            pl.BlockSpec((NDOC, NT), lambda i: (0, 0)),
            pl.BlockSpec((NWID + D_IN, NT), lambda i: (0, 0)),
            pl.BlockSpec((1, NT), lambda i: (0, 0)),
        ],
        out_specs=pl.BlockSpec((BLK, NT), lambda i: (i, 0)),
        out_shape=jax.ShapeDtypeStruct((NP, NT), jnp.float32),
    )


def kernel(idx_x, idx_w, x_batch, edge_index, edge_weight, word_vec,
           W_rel, b_rel, W_root, bn1_gamma, bn1_beta,
           W_fc1, b_fc1, W_fc2, b_fc2,
           W_mean, b_mean, bn_mean_gamma, bn_mean_beta,
           W_logvar, b_logvar, W_phi, b_phi):
    f32 = jnp.float32
    idx_pad = jnp.pad(idx_x.astype(jnp.int32), (0, NP - N))
    src = jnp.pad(edge_index[0].astype(jnp.int32), (0, EP - E))
    dst = jnp.pad(edge_index[1].astype(jnp.int32), (0, EP - E))
    ew_pad = jnp.pad(edge_weight, (0, EP - E))
    epk = jnp.pad(jnp.stack([src.reshape(-1, EDGE_K), dst.reshape(-1, EDGE_K)],
                            axis=1),
                  ((0, 2), (0, 0), (0, 0)))
    ew_rep = jnp.pad(
        jnp.broadcast_to(ew_pad.reshape(EP // EDGE_K, EDGE_K, 1),
                         (EP // EDGE_K, EDGE_K, 16)),
        ((0, 2), (0, 0), (0, 0)))
    iw = jnp.pad(idx_w, (0, NP - N)).reshape(NP, 1)
    xb = jnp.pad(x_batch, (0, NP - N), constant_values=NDOC).reshape(NP, 1)
    zeros_half = jnp.zeros((NP, 1, 128), f32)

    x = _make_sc_gather(50000, D_IN, NP, 160)(word_vec, idx_pad)
    agg = _make_sc_edge()(x.reshape(NP, 2, 128), epk, ew_rep,
                          zeros_half).reshape(NP, D_IN)

    h, stats = _tc_h()(
        agg, x, iw, W_rel, W_root, b_rel.reshape(1, NWID))
    enc1, doc = _tc_enc()(
        h, stats, x, xb, bn1_gamma.reshape(1, NWID), bn1_beta.reshape(1, NWID),
        W_fc1, b_fc1.reshape(1, ENC_NH), W_fc2, b_fc2.reshape(1, ENC_NH))
    mean, logvar, docproj = _tc_doc()(
        doc, W_mean, b_mean.reshape(1, NT), bn_mean_gamma.reshape(1, NT),
        bn_mean_beta.reshape(1, NT), W_logvar, b_logvar.reshape(1, NT),
        W_phi[NWID + D_IN:, :])
    phi = _tc_phi()(enc1, xb, docproj, W_phi[:NWID + D_IN, :],
                    b_phi.reshape(1, NT))
    return (mean, logvar, phi[:N])
